# baseline (speedup 1.0000x reference)
import functools
import jax
import jax.numpy as jnp
from jax import lax
from jax.experimental import pallas as pl
from jax.experimental.pallas import tpu as pltpu
from jax.experimental.pallas import tpu_sc as plsc

F32 = jnp.float32
I32 = jnp.int32
_BN = 1000
_BE = 2000

_NPAD = 50048
_SPS = 3128
_GK = 500
_SK = 2000

_sc_mesh = plsc.VectorSubcoreMesh(core_axis_name="c", subcore_axis_name="s")


def _sc_gather2(ta, tb, dst_idx, src_idx):
    e = dst_idx.shape[0]
    epw = e // 32

    @functools.partial(
        pl.kernel, mesh=_sc_mesh,
        out_type=[jax.ShapeDtypeStruct((e, 64), F32),
                  jax.ShapeDtypeStruct((e, 64), F32)],
        scratch_types=[pltpu.VMEM((_GK,), I32), pltpu.VMEM((_GK, 64), F32),
                       pltpu.VMEM((_GK,), I32), pltpu.VMEM((_GK, 64), F32),
                       pltpu.SemaphoreType.DMA, pltpu.SemaphoreType.DMA],
    )
    def k(ta_h, tb_h, di_h, si_h, oa_h, ob_h, iv1, rv1, iv2, rv2, s1, s2):
        wid = lax.axis_index("s") * 2 + lax.axis_index("c")
        base = wid * epw

        def body(i, carry):
            off = base + i * _GK
            pltpu.sync_copy(di_h.at[pl.ds(off, _GK)], iv1)
            pltpu.sync_copy(si_h.at[pl.ds(off, _GK)], iv2)
            c1 = pltpu.async_copy(ta_h.at[iv1], rv1, s1)
            c2 = pltpu.async_copy(tb_h.at[iv2], rv2, s2)
            c1.wait()
            c2.wait()
            pltpu.sync_copy(rv1, oa_h.at[pl.ds(off, _GK)])
            pltpu.sync_copy(rv2, ob_h.at[pl.ds(off, _GK)])
            return carry

        lax.fori_loop(0, epw // _GK, body, 0)

    return k(ta, tb, dst_idx, src_idx)


def _zero_rows(buf, rows, cols):
    z = jnp.zeros((16,), F32)

    def body(i, carry):
        for c0 in range(0, cols, 16):
            buf[i, c0:c0 + 16] = z
        return carry

    lax.fori_loop(0, rows, body, 0)


def _sc_segsum64(tstack, src_idx, didx):
    e = src_idx.shape[0]
    eps = e // 16

    @functools.partial(
        pl.kernel, mesh=_sc_mesh,
        out_type=jax.ShapeDtypeStruct((2, _NPAD, 32), F32),
        scratch_types=[pltpu.VMEM((_SK,), I32), pltpu.VMEM((_SK,), I32),
                       pltpu.VMEM((_SK, 32), F32),
                       pltpu.VMEM_SHARED((_NPAD, 32), F32),
                       pltpu.SemaphoreType.DMA],
    )
    def k(tb_h, si_h, di_h, out_h, siv, div, rv, acc, sem):
        cid = lax.axis_index("c")
        sid = lax.axis_index("s")
        _zero_rows(rv, _SK, 32)
        rowbase = sid * _SPS
        pltpu.sync_copy(rv, acc.at[pl.ds(rowbase, _SK)])
        pltpu.sync_copy(rv.at[pl.ds(0, _SPS - _SK)],
                        acc.at[pl.ds(rowbase + _SK, _SPS - _SK)])
        plsc.subcore_barrier()

        base = sid * eps

        def body(i, carry):
            off = base + i * _SK
            pltpu.sync_copy(si_h.at[pl.ds(off, _SK)], siv)
            pltpu.sync_copy(di_h.at[pl.ds(off, _SK)], div)
            pltpu.async_copy(tb_h.at[cid].at[siv], rv, sem).wait()
            pltpu.sync_copy(rv, acc.at[div], add=True)
            return carry

        lax.fori_loop(0, eps // _SK, body, 0)
        plsc.subcore_barrier()
        pltpu.sync_copy(acc.at[pl.ds(rowbase, _SPS)],
                        out_h.at[cid].at[pl.ds(rowbase, _SPS)])

    return k(tstack, src_idx, didx)


def _sc_segsum16(t16, src_idx, didx):
    e = src_idx.shape[0]
    eps = e // 32

    @functools.partial(
        pl.kernel, mesh=_sc_mesh,
        out_type=jax.ShapeDtypeStruct((2, _NPAD, 16), F32),
        scratch_types=[pltpu.VMEM((_SK,), I32), pltpu.VMEM((_SK,), I32),
                       pltpu.VMEM((_SK, 16), F32),
                       pltpu.VMEM_SHARED((_NPAD, 16), F32),
                       pltpu.SemaphoreType.DMA],
    )
    def k(tb_h, si_h, di_h, out_h, siv, div, rv, acc, sem):
        cid = lax.axis_index("c")
        sid = lax.axis_index("s")
        _zero_rows(rv, _SK, 16)
        rowbase = sid * _SPS
        pltpu.sync_copy(rv, acc.at[pl.ds(rowbase, _SK)])
        pltpu.sync_copy(rv.at[pl.ds(0, _SPS - _SK)],
                        acc.at[pl.ds(rowbase + _SK, _SPS - _SK)])
        plsc.subcore_barrier()

        base = (cid * 16 + sid) * eps

        def body(i, carry):
            off = base + i * _SK
            pltpu.sync_copy(si_h.at[pl.ds(off, _SK)], siv)
            pltpu.sync_copy(di_h.at[pl.ds(off, _SK)], div)
            pltpu.async_copy(tb_h.at[siv], rv, sem).wait()
            pltpu.sync_copy(rv, acc.at[div], add=True)
            return carry

        lax.fori_loop(0, eps // _SK, body, 0)
        plsc.subcore_barrier()
        pltpu.sync_copy(acc.at[pl.ds(rowbase, _SPS)],
                        out_h.at[cid].at[pl.ds(rowbase, _SPS)])

    return k(t16, src_idx, didx)


def _sc_counts(d1, d2, d3):
    e = d1.shape[0]
    eps = e // 16

    @functools.partial(
        pl.kernel, mesh=_sc_mesh,
        out_type=jax.ShapeDtypeStruct((3, _NPAD, 16), F32),
        scratch_types=[pltpu.VMEM((_SK,), I32), pltpu.VMEM((_SK, 16), F32),
                       pltpu.VMEM((_SK, 16), F32),
                       pltpu.VMEM_SHARED((_NPAD, 16), F32),
                       pltpu.SemaphoreType.DMA],
    )
    def k(d1_h, d2_h, d3_h, out_h, div, ones, zrows, acc, sem):
        cid = lax.axis_index("c")
        sid = lax.axis_index("s")
        _zero_rows(zrows, _SK, 16)
        e0 = jnp.where(lax.iota(I32, 16) == 0, 1.0, 0.0).astype(F32)

        def fill(i, carry):
            ones[i, 0:16] = e0
            return carry

        lax.fori_loop(0, _SK, fill, 0)
        rowbase = sid * _SPS
        base = sid * eps

        def zero_acc():
            pltpu.sync_copy(zrows, acc.at[pl.ds(rowbase, _SK)])
            pltpu.sync_copy(zrows.at[pl.ds(0, _SPS - _SK)],
                            acc.at[pl.ds(rowbase + _SK, _SPS - _SK)])
            plsc.subcore_barrier()

        def scatter_loop(d_h):
            def body(i, carry):
                off = base + i * _SK
                pltpu.sync_copy(d_h.at[pl.ds(off, _SK)], div)
                pltpu.sync_copy(ones, acc.at[div], add=True)
                return carry

            lax.fori_loop(0, eps // _SK, body, 0)

        zero_acc()

        @pl.when(cid == 0)
        def _():
            scatter_loop(d1_h)

        @pl.when(cid == 1)
        def _():
            scatter_loop(d2_h)

        plsc.subcore_barrier()
        pltpu.sync_copy(acc.at[pl.ds(rowbase, _SPS)],
                        out_h.at[cid].at[pl.ds(rowbase, _SPS)])
        plsc.subcore_barrier()

        zero_acc()

        @pl.when(cid == 0)
        def _():
            scatter_loop(d3_h)

        plsc.subcore_barrier()

        @pl.when(cid == 0)
        def _():
            pltpu.sync_copy(acc.at[pl.ds(rowbase, _SPS)],
                            out_h.at[2].at[pl.ds(rowbase, _SPS)])

    return k(d1, d2, d3)


def _k_proj(x_ref, w_ref, b_ref, z_ref, ssum_ref, ssq_ref):
    i = pl.program_id(0)
    xb = x_ref[...]
    w = w_ref[...]
    z = (jnp.dot(xb[:, :1024], w[:1024], preferred_element_type=F32)
         + jnp.dot(xb[:, 1024:], w[1024:], preferred_element_type=F32)
         + b_ref[...])
    z_ref[...] = z

    @pl.when(i == 0)
    def _():
        ssum_ref[...] = jnp.zeros_like(ssum_ref)
        ssq_ref[...] = jnp.zeros_like(ssq_ref)

    ssum_ref[...] += jnp.broadcast_to(z.sum(0), ssum_ref.shape)
    ssq_ref[...] += jnp.broadcast_to((z * z).sum(0), ssq_ref.shape)


def _k_tables(z_ref, sc_ref, sh_ref, w_ref, b_ref, out_ref):
    h = jax.nn.relu(z_ref[...] * sc_ref[...] + sh_ref[...])
    out_ref[...] = jnp.dot(h, w_ref[...], preferred_element_type=F32) + b_ref[...]


def _k_edge_mlp(a_ref, b_ref, w_ref, bias_ref, out_ref):
    t = jax.nn.relu(a_ref[...] + b_ref[...])
    out_ref[...] = jnp.dot(t, w_ref[...], preferred_element_type=F32) + bias_ref[...]


def _k_fix_stats(agg_ref, ssum_ref, ssq_ref):
    i = pl.program_id(0)
    a = agg_ref[...]
    a = jnp.where(jnp.isfinite(a), a, 0.0)

    @pl.when(i == 0)
    def _():
        ssum_ref[...] = jnp.zeros_like(ssum_ref)
        ssq_ref[...] = jnp.zeros_like(ssq_ref)

    ssum_ref[...] += jnp.broadcast_to(a.sum(0), ssum_ref.shape)
    ssq_ref[...] += jnp.broadcast_to((a * a).sum(0), ssq_ref.shape)


def _k_fix_apply(agg_ref, sc_ref, sh_ref, w_ref, out_ref):
    a = agg_ref[...]
    a = jnp.where(jnp.isfinite(a), a, 0.0)
    y = jax.nn.relu(a * sc_ref[...] + sh_ref[...])
    out_ref[...] = jnp.dot(y, w_ref[...], preferred_element_type=F32)


def _k_sage_stats(sm_ref, cnt_ref, r_ref, bl_ref, ssum_ref, ssq_ref):
    i = pl.program_id(0)
    c = jnp.maximum(cnt_ref[...], 1.0)
    t = sm_ref[...] / c + bl_ref[...] + r_ref[...]

    @pl.when(i == 0)
    def _():
        ssum_ref[...] = jnp.zeros_like(ssum_ref)
        ssq_ref[...] = jnp.zeros_like(ssq_ref)

    ssum_ref[...] += jnp.broadcast_to(t.sum(0), ssum_ref.shape)
    ssq_ref[...] += jnp.broadcast_to((t * t).sum(0), ssq_ref.shape)


def _k_sage_apply(sm_ref, cnt_ref, r_ref, bl_ref, sc_ref, sh_ref, w_ref, out_ref):
    c = jnp.maximum(cnt_ref[...], 1.0)
    t = sm_ref[...] / c + bl_ref[...] + r_ref[...]
    z2 = jax.nn.relu(t * sc_ref[...] + sh_ref[...])
    out_ref[...] = jnp.dot(z2, w_ref[...], preferred_element_type=F32)


def _k_final(s_ref, c_ref, v_ref, bl_ref, out_ref):
    t = s_ref[...] / jnp.maximum(c_ref[...], 1.0) + v_ref[...] + bl_ref[...]
    lanes = jax.lax.broadcasted_iota(jnp.int32, t.shape, 1)
    t = jnp.where(lanes < 3, t, 0.0)
    out_ref[...] = jax.nn.sigmoid(t.sum(axis=1, keepdims=True))


def _row_spec(blk, ncols):
    return pl.BlockSpec((blk, ncols), lambda i: (i, 0))


def _full_spec(shape):
    return pl.BlockSpec(shape, lambda i: (0, 0))


def _stats_to_affine(ssum, ssq, g, be, n):
    m = ssum[0] / n
    v = ssq[0] / n - m * m
    scale = g / jnp.sqrt(v + 1e-5)
    shift = be - m * scale
    return scale[None, :], shift[None, :]


def kernel(x, edge_index, edge_attr, params):
    p = params
    n, feat4 = x.shape
    e = edge_index.shape[1]
    src, dst = edge_index[0], edge_index[1]
    sel1 = edge_attr >= 0
    sel2 = edge_attr <= 0
    d1 = jnp.where(sel1, dst, n)
    d2 = jnp.where(sel2, dst, n)
    dmask = (d1, d2, dst)

    wbig = jnp.concatenate([
        p['W012'], p['W011'][:512], p['W_spf'] @ p['W011'][512:]], axis=0)
    bbig = (p['b012'] + p['b011'] + p['b_spf'] @ p['W011'][512:])[None, :]

    grid_n = n // _BN
    grid_e = e // _BE

    z, ssum, ssq = pl.pallas_call(
        _k_proj,
        grid=(grid_n,),
        in_specs=[_row_spec(_BN, feat4), _full_spec((feat4, 64)), _full_spec((1, 64))],
        out_specs=[_row_spec(_BN, 64), _full_spec((8, 64)), _full_spec((8, 64))],
        out_shape=[jax.ShapeDtypeStruct((n, 64), F32),
                   jax.ShapeDtypeStruct((8, 64), F32),
                   jax.ShapeDtypeStruct((8, 64), F32)],
    )(x, wbig, bbig)
    sc0, sh0 = _stats_to_affine(ssum, ssq, p['g01'], p['be01'], n)

    wcat = []
    bcat = []
    for tag in ('11', '12', '13'):
        w1 = p['W%sa' % tag]
        wcat += [w1[:64] - w1[64:], w1[64:]]
        bcat += [p['b%sa' % tag], jnp.zeros((64,), F32)]
    wcat = jnp.concatenate(wcat, axis=1)
    bcat = jnp.concatenate(bcat)[None, :]

    tables = pl.pallas_call(
        _k_tables,
        grid=(grid_n,),
        in_specs=[_row_spec(_BN, 64), _full_spec((1, 64)), _full_spec((1, 64)),
                  _full_spec((64, 384)), _full_spec((1, 384))],
        out_specs=_row_spec(_BN, 384),
        out_shape=jax.ShapeDtypeStruct((n, 384), F32),
    )(z, sc0, sh0, wcat, bcat)

    ones = jnp.ones((e,), F32)
    cnt = [jax.ops.segment_sum(ones, d, num_segments=n)[:, None] for d in dmask]

    wlr21 = jnp.concatenate([p['Wl21'], p['Wr21']], axis=1)
    bl21 = p['bl21'][None, :]

    su_list, v_list = [], []
    for b, (tag, otag) in enumerate((('11', '31'), ('12', '32'), ('13', '33'))):
        adst = tables[:, 128 * b:128 * b + 64][dst]
        bsrc = tables[:, 128 * b + 64:128 * b + 128][src]
        he = pl.pallas_call(
            _k_edge_mlp,
            grid=(grid_e,),
            in_specs=[_row_spec(_BE, 64), _row_spec(_BE, 64),
                      _full_spec((64, 64)), _full_spec((1, 64))],
            out_specs=_row_spec(_BE, 64),
            out_shape=jax.ShapeDtypeStruct((e, 64), F32),
        )(adst, bsrc, p['W%sb' % tag], p['b%sb' % tag][None, :])

        agg = jax.ops.segment_max(he, dmask[b], num_segments=n)

        s1, q1 = pl.pallas_call(
            _k_fix_stats,
            grid=(grid_n,),
            in_specs=[_row_spec(_BN, 64)],
            out_specs=[_full_spec((8, 64)), _full_spec((8, 64))],
            out_shape=[jax.ShapeDtypeStruct((8, 64), F32),
                       jax.ShapeDtypeStruct((8, 64), F32)],
        )(agg)
        scb, shb = _stats_to_affine(s1, q1, p['g' + tag], p['be' + tag], n)

        yt = pl.pallas_call(
            _k_fix_apply,
            grid=(grid_n,),
            in_specs=[_row_spec(_BN, 64), _full_spec((1, 64)), _full_spec((1, 64)),
                      _full_spec((64, 128))],
            out_specs=_row_spec(_BN, 128),
            out_shape=jax.ShapeDtypeStruct((n, 128), F32),
        )(agg, scb, shb, wlr21)

        sm = jax.ops.segment_sum(yt[:, :64][src], dmask[b], num_segments=n)

        s2, q2 = pl.pallas_call(
            _k_sage_stats,
            grid=(grid_n,),
            in_specs=[_row_spec(_BN, 64), _row_spec(_BN, 1), _row_spec(_BN, 64),
                      _full_spec((1, 64))],
            out_specs=[_full_spec((8, 64)), _full_spec((8, 64))],
            out_shape=[jax.ShapeDtypeStruct((8, 64), F32),
                       jax.ShapeDtypeStruct((8, 64), F32)],
        )(sm, cnt[b], yt[:, 64:], bl21)
        sc2, sh2 = _stats_to_affine(s2, q2, p['g21'], p['be21'], n)

        wuv = jnp.zeros((64, 16), F32).at[:, 0].set(p['Wl' + otag][:, 0]) \
                                      .at[:, 1].set(p['Wr' + otag][:, 0])
        uv = pl.pallas_call(
            _k_sage_apply,
            grid=(grid_n,),
            in_specs=[_row_spec(_BN, 64), _row_spec(_BN, 1), _row_spec(_BN, 64),
                      _full_spec((1, 64)), _full_spec((1, 64)), _full_spec((1, 64)),
                      _full_spec((64, 16))],
            out_specs=_row_spec(_BN, 16),
            out_shape=jax.ShapeDtypeStruct((n, 16), F32),
        )(sm, cnt[b], yt[:, 64:], bl21, sc2, sh2, wuv)

        su = jax.ops.segment_sum(uv[:, 0][src], dmask[b], num_segments=n)
        su_list.append(su)
        v_list.append(uv[:, 1] + p['bl' + otag][0])

    spack = jnp.stack(su_list + [jnp.zeros((n,), F32)] * 5, axis=1)
    cpack = jnp.concatenate(cnt + [jnp.ones((n, 1), F32)] * 5, axis=1)
    vpack = jnp.stack(v_list + [jnp.zeros((n,), F32)] * 5, axis=1)

    out = pl.pallas_call(
        _k_final,
        grid=(grid_n,),
        in_specs=[_row_spec(_BN, 8), _row_spec(_BN, 8), _row_spec(_BN, 8),
                  _full_spec((1, 8))],
        out_specs=_row_spec(_BN, 1),
        out_shape=jax.ShapeDtypeStruct((n, 1), F32),
    )(spack, cpack, vpack, jnp.zeros((1, 8), F32))
    return out


# speedup vs baseline: 2.0823x; 2.0823x over previous
import jax
import jax.numpy as jnp
from jax.experimental import pallas as pl
from jax.experimental.pallas import tpu as pltpu

F32 = jnp.float32
_BN = 1000
_BE = 2000


def _k_proj(x_ref, w_ref, b_ref, z_ref, ssum_ref, ssq_ref):
    i = pl.program_id(0)
    xb = x_ref[...]
    w = w_ref[...]
    z = (jnp.dot(xb[:, :1024], w[:1024], preferred_element_type=F32)
         + jnp.dot(xb[:, 1024:], w[1024:], preferred_element_type=F32)
         + b_ref[...])
    z_ref[...] = z

    @pl.when(i == 0)
    def _():
        ssum_ref[...] = jnp.zeros_like(ssum_ref)
        ssq_ref[...] = jnp.zeros_like(ssq_ref)

    ssum_ref[...] += jnp.broadcast_to(z.sum(0), ssum_ref.shape)
    ssq_ref[...] += jnp.broadcast_to((z * z).sum(0), ssq_ref.shape)


def _k_tables(z_ref, sc_ref, sh_ref, w_ref, b_ref, out_ref):
    h = jax.nn.relu(z_ref[...] * sc_ref[...] + sh_ref[...])
    out_ref[...] = jnp.dot(h, w_ref[...], preferred_element_type=F32) + b_ref[...]


def _k_edge_mlp(a_ref, b_ref, m_ref, w_ref, bias_ref, out_ref):
    a = a_ref[...]
    b = b_ref[...]
    msk = m_ref[...]
    w = w_ref[...]
    bias = bias_ref[...]
    for i in range(3):
        t = jax.nn.relu(a[:, 64 * i:64 * i + 64] + b[:, 64 * i:64 * i + 64])
        he = (jnp.dot(t, w[64 * i:64 * i + 64], preferred_element_type=F32)
              + bias[:, 64 * i:64 * i + 64])
        out_ref[:, 64 * i:64 * i + 64] = he + msk[:, i:i + 1]


def _k_fix_stats(agg_ref, ssum_ref, ssq_ref):
    i = pl.program_id(0)
    a = agg_ref[...]
    a = jnp.where(jnp.isfinite(a), a, 0.0)

    @pl.when(i == 0)
    def _():
        ssum_ref[...] = jnp.zeros_like(ssum_ref)
        ssq_ref[...] = jnp.zeros_like(ssq_ref)

    ssum_ref[...] += jnp.broadcast_to(a.sum(0), ssum_ref.shape)
    ssq_ref[...] += jnp.broadcast_to((a * a).sum(0), ssq_ref.shape)


def _k_fix_apply(agg_ref, sc_ref, sh_ref, wl_ref, wr_ref, out_ref):
    a = agg_ref[...]
    a = jnp.where(jnp.isfinite(a), a, 0.0)
    y = jax.nn.relu(a * sc_ref[...] + sh_ref[...])
    wl = wl_ref[...]
    wr = wr_ref[...]
    for i in range(3):
        yb = y[:, 64 * i:64 * i + 64]
        out_ref[:, 64 * i:64 * i + 64] = jnp.dot(
            yb, wl, preferred_element_type=F32)
        out_ref[:, 192 + 64 * i:192 + 64 * i + 64] = jnp.dot(
            yb, wr, preferred_element_type=F32)


def _k_sage_stats(sm_ref, cnt_ref, r_ref, bl_ref, ssum_ref, ssq_ref):
    i = pl.program_id(0)
    sm = sm_ref[...]
    cz = cnt_ref[...]
    r = r_ref[...]
    bl = bl_ref[...]
    t = jnp.concatenate(
        [sm[:, 64 * b:64 * b + 64] / jnp.maximum(cz[:, b:b + 1], 1.0)
         + bl + r[:, 64 * b:64 * b + 64] for b in range(3)], axis=1)

    @pl.when(i == 0)
    def _():
        ssum_ref[...] = jnp.zeros_like(ssum_ref)
        ssq_ref[...] = jnp.zeros_like(ssq_ref)

    ssum_ref[...] += jnp.broadcast_to(t.sum(0), ssum_ref.shape)
    ssq_ref[...] += jnp.broadcast_to((t * t).sum(0), ssq_ref.shape)


def _k_sage_apply(sm_ref, cnt_ref, r_ref, bl_ref, sc_ref, sh_ref, w_ref, out_ref):
    sm = sm_ref[...]
    cz = cnt_ref[...]
    r = r_ref[...]
    bl = bl_ref[...]
    t = jnp.concatenate(
        [sm[:, 64 * b:64 * b + 64] / jnp.maximum(cz[:, b:b + 1], 1.0)
         + bl + r[:, 64 * b:64 * b + 64] for b in range(3)], axis=1)
    z2 = jax.nn.relu(t * sc_ref[...] + sh_ref[...])
    out_ref[...] = jnp.dot(z2, w_ref[...], preferred_element_type=F32)


def _k_final(s_ref, c_ref, v_ref, bl_ref, out_ref):
    t = s_ref[...] / jnp.maximum(c_ref[...], 1.0) + v_ref[...] + bl_ref[...]
    lanes = jax.lax.broadcasted_iota(jnp.int32, t.shape, 1)
    t = jnp.where(lanes < 3, t, 0.0)
    out_ref[...] = jax.nn.sigmoid(t.sum(axis=1, keepdims=True))


def _row_spec(blk, ncols):
    return pl.BlockSpec((blk, ncols), lambda i: (i, 0))


def _full_spec(shape):
    return pl.BlockSpec(shape, lambda i: (0, 0))


def _stats_to_affine(ssum, ssq, g, be, n):
    m = ssum[0] / n
    v = ssq[0] / n - m * m
    scale = g / jnp.sqrt(v + 1e-5)
    shift = be - m * scale
    return scale[None, :], shift[None, :]


def kernel(x, edge_index, edge_attr, params):
    p = params
    n, feat4 = x.shape
    e = edge_index.shape[1]
    src, dst = edge_index[0], edge_index[1]
    s1f = (edge_attr >= 0).astype(F32)[:, None]
    s2f = (edge_attr <= 0).astype(F32)[:, None]

    neg = jnp.float32(-jnp.inf)
    mask3 = jnp.concatenate(
        [jnp.where(s1f > 0, 0.0, neg), jnp.where(s2f > 0, 0.0, neg),
         jnp.zeros((e, 6), F32)], axis=1)
    selcnt = jnp.concatenate(
        [s1f, s2f, jnp.ones((e, 1), F32), jnp.zeros((e, 5), F32)], axis=1)

    wbig = jnp.concatenate([
        p['W012'], p['W011'][:512], p['W_spf'] @ p['W011'][512:]], axis=0)
    bbig = (p['b012'] + p['b011'] + p['b_spf'] @ p['W011'][512:])[None, :]

    grid_n = n // _BN
    grid_e = e // _BE

    z, ssum, ssq = pl.pallas_call(
        _k_proj,
        grid=(grid_n,),
        in_specs=[_row_spec(_BN, feat4), _full_spec((feat4, 64)), _full_spec((1, 64))],
        out_specs=[_row_spec(_BN, 64), _full_spec((8, 64)), _full_spec((8, 64))],
        out_shape=[jax.ShapeDtypeStruct((n, 64), F32),
                   jax.ShapeDtypeStruct((8, 64), F32),
                   jax.ShapeDtypeStruct((8, 64), F32)],
    )(x, wbig, bbig)
    sc0, sh0 = _stats_to_affine(ssum, ssq, p['g01'], p['be01'], n)

    wa, ba, wb2, b2 = [], [], [], []
    for tag in ('11', '12', '13'):
        w1 = p['W%sa' % tag]
        wa.append(w1[:64] - w1[64:])
        ba.append(p['b%sa' % tag])
        wb2.append(p['W%sb' % tag])
        b2.append(p['b%sb' % tag])
    wcat = jnp.concatenate(wa + [p['W%sa' % t][64:] for t in ('11', '12', '13')],
                           axis=1)
    bcat = jnp.concatenate(ba + [jnp.zeros((192,), F32)])[None, :]
    w2cat = jnp.concatenate(wb2, axis=0)
    b2cat = jnp.concatenate(b2)[None, :]

    tables = pl.pallas_call(
        _k_tables,
        grid=(grid_n,),
        in_specs=[_row_spec(_BN, 64), _full_spec((1, 64)), _full_spec((1, 64)),
                  _full_spec((64, 384)), _full_spec((1, 384))],
        out_specs=_row_spec(_BN, 384),
        out_shape=jax.ShapeDtypeStruct((n, 384), F32),
    )(z, sc0, sh0, wcat, bcat)

    adst = tables[:, :192][dst]
    bsrc = tables[:, 192:][src]
    he = pl.pallas_call(
        _k_edge_mlp,
        grid=(grid_e,),
        in_specs=[_row_spec(_BE, 192), _row_spec(_BE, 192), _row_spec(_BE, 8),
                  _full_spec((192, 64)), _full_spec((1, 192))],
        out_specs=_row_spec(_BE, 192),
        out_shape=jax.ShapeDtypeStruct((e, 192), F32),
    )(adst, bsrc, mask3, w2cat, b2cat)

    agg = jax.ops.segment_max(he, dst, num_segments=n)

    s1, q1 = pl.pallas_call(
        _k_fix_stats,
        grid=(grid_n,),
        in_specs=[_row_spec(_BN, 192)],
        out_specs=[_full_spec((8, 192)), _full_spec((8, 192))],
        out_shape=[jax.ShapeDtypeStruct((8, 192), F32),
                   jax.ShapeDtypeStruct((8, 192), F32)],
    )(agg)
    gcat = jnp.concatenate([p['g11'], p['g12'], p['g13']])
    becat = jnp.concatenate([p['be11'], p['be12'], p['be13']])
    scb, shb = _stats_to_affine(s1, q1, gcat, becat, n)

    yt = pl.pallas_call(
        _k_fix_apply,
        grid=(grid_n,),
        in_specs=[_row_spec(_BN, 192), _full_spec((1, 192)), _full_spec((1, 192)),
                  _full_spec((64, 64)), _full_spec((64, 64))],
        out_specs=_row_spec(_BN, 384),
        out_shape=jax.ShapeDtypeStruct((n, 384), F32),
    )(agg, scb, shb, p['Wl21'], p['Wr21'])

    ytl_g = yt[:, :192][src]
    mul = jnp.concatenate([jnp.broadcast_to(s1f, (e, 64)),
                           jnp.broadcast_to(s2f, (e, 64)),
                           jnp.ones((e, 64), F32)], axis=1)
    vals = jnp.concatenate([ytl_g * mul, selcnt], axis=1)
    smc = jax.ops.segment_sum(vals, dst, num_segments=n)
    sm = smc[:, :192]
    cntc = smc[:, 192:200]

    bl21 = p['bl21'][None, :]
    s2, q2 = pl.pallas_call(
        _k_sage_stats,
        grid=(grid_n,),
        in_specs=[_row_spec(_BN, 192), _row_spec(_BN, 8), _row_spec(_BN, 192),
                  _full_spec((1, 64))],
        out_specs=[_full_spec((8, 192)), _full_spec((8, 192))],
        out_shape=[jax.ShapeDtypeStruct((8, 192), F32),
                   jax.ShapeDtypeStruct((8, 192), F32)],
    )(sm, cntc, yt[:, 192:], bl21)
    g2cat = jnp.concatenate([p['g21']] * 3)
    be2cat = jnp.concatenate([p['be21']] * 3)
    sc2, sh2 = _stats_to_affine(s2, q2, g2cat, be2cat, n)

    wuv = jnp.zeros((192, 16), F32)
    for b, otag in enumerate(('31', '32', '33')):
        wuv = wuv.at[64 * b:64 * b + 64, b].set(p['Wl' + otag][:, 0])
        wuv = wuv.at[64 * b:64 * b + 64, 3 + b].set(p['Wr' + otag][:, 0])

    uv = pl.pallas_call(
        _k_sage_apply,
        grid=(grid_n,),
        in_specs=[_row_spec(_BN, 192), _row_spec(_BN, 8), _row_spec(_BN, 192),
                  _full_spec((1, 64)), _full_spec((1, 192)), _full_spec((1, 192)),
                  _full_spec((192, 16))],
        out_specs=_row_spec(_BN, 16),
        out_shape=jax.ShapeDtypeStruct((n, 16), F32),
    )(sm, cntc, yt[:, 192:], bl21, sc2, sh2, wuv)

    sel16 = jnp.concatenate([s1f, s2f, jnp.ones((e, 14), F32)], axis=1)
    su = jax.ops.segment_sum(uv[src] * sel16, dst, num_segments=n)

    spack = su[:, :8]
    cpack = jnp.concatenate([cntc[:, :3], jnp.ones((n, 5), F32)], axis=1)
    vpack = uv[:, 3:11]
    bladd = jnp.zeros((1, 8), F32)
    for b, otag in enumerate(('31', '32', '33')):
        bladd = bladd.at[0, b].set(p['bl' + otag][0])

    out = pl.pallas_call(
        _k_final,
        grid=(grid_n,),
        in_specs=[_row_spec(_BN, 8), _row_spec(_BN, 8), _row_spec(_BN, 8),
                  _full_spec((1, 8))],
        out_specs=_row_spec(_BN, 1),
        out_shape=jax.ShapeDtypeStruct((n, 1), F32),
    )(spack, cpack, vpack, bladd)
    return out
